# baseline (speedup 1.0000x reference)
import functools

import jax
import jax.numpy as jnp
from jax import lax
from jax.experimental import pallas as pl
from jax.experimental.pallas import tpu as pltpu
from jax.experimental.pallas import tpu_sc as plsc

N_SMPS = 16384
D_IN = 128
N_LAYERS = 5
GROWTH = 5
D_EMBED = 2
N_FCNS = 250
K = 5
D_OUT = 2

HW = 160
NFP = 256
BLK = 8192

NW = 32
LQ = 16
SPL = NFP * LQ
NHALF = N_SMPS // 2


def _dot(a, b, precision=None, dims=(((1,), (0,)), ((), ()))):
    return jax.lax.dot_general(a, b, dims, precision=precision,
                               preferred_element_type=jnp.float32)


def _dense_body(x_ref, w0_ref, w1_ref, w2_ref, w3_ref, w4_ref, ow_ref, ob_ref,
                ctrs_ref, xe8_ref, xet_ref, ctspl_ref, hbuf, wcat):
    nb = x_ref.shape[0]
    bf = jnp.bfloat16
    f32 = jnp.float32

    @pl.when(pl.program_id(0) == 0)
    def _():
        wcat[:, :] = jnp.zeros((HW, 48), f32)
        wcat[:D_IN, 0:GROWTH] = w0_ref[:]
        wcat[:D_IN + GROWTH, 8:8 + GROWTH] = w1_ref[:]
        wcat[:D_IN + 2 * GROWTH, 16:16 + GROWTH] = w2_ref[:]
        wcat[:D_IN + 3 * GROWTH, 24:24 + GROWTH] = w3_ref[:]
        wcat[:D_IN + 4 * GROWTH, 32:32 + GROWTH] = w4_ref[:]
        wcat[:D_IN + 5 * GROWTH, 40:40 + D_EMBED] = ow_ref[:]

    hbuf[:, :D_IN] = x_ref[:]
    hbuf[:, D_IN:] = jnp.zeros((nb, HW - D_IN), f32)
    for i in range(N_LAYERS):
        w = wcat[:, 8 * i:8 * i + 8]
        z = _dot(hbuf[:].astype(bf), w.astype(bf))
        t = jnp.tanh(z).astype(bf).astype(f32)
        if i == 0:
            hbuf[:, :D_IN] = x_ref[:].astype(bf).astype(f32)
        hbuf[:, D_IN + GROWTH * i:D_IN + GROWTH * i + 8] = t
    xe2 = _dot(hbuf[:].astype(bf), wcat[:, 40:48].astype(bf))
    xe2 = xe2 + jnp.concatenate(
        [ob_ref[:], jnp.zeros((1, 6), f32)], axis=1)
    xe0 = xe2[:, 0:1]
    xe1 = xe2[:, 1:2]
    xe0h = xe0.astype(bf).astype(f32)
    xe1h = xe1.astype(bf).astype(f32)
    zc = jnp.zeros((nb, 1), f32)
    xe8 = jnp.concatenate([xe0, xe1, xe0h, xe1h, zc, zc, zc, zc], axis=1)
    xe8_ref[:] = xe8
    xet_ref[:] = jax.lax.transpose(xe8, (1, 0))

    @pl.when(pl.program_id(0) == 0)
    def _():
        cT = ctrs_ref[:]
        c0 = cT[:, 0:1]
        c1 = cT[:, 1:2]
        sc = c0 * c0 + c1 * c1
        cols = jnp.concatenate(
            [-2.0 * c0.astype(bf).astype(f32),
             -2.0 * c1.astype(bf).astype(f32), sc],
            axis=1)
        pad = jnp.concatenate(
            [jnp.zeros((NFP - N_FCNS, 2), f32),
             jnp.full((NFP - N_FCNS, 1), jnp.inf, f32)], axis=1)
        colsp = jnp.concatenate([cols, pad], axis=0)
        ctspl_ref[:] = jnp.concatenate(
            [jnp.broadcast_to(colsp[:, p:p + 1], (NFP, LQ)) for p in range(3)],
            axis=0)


def _make_sc_body(nrows):
    rpw = nrows // NW
    ng = rpw // LQ

    def _sc_body(xe_hbm, ct_hbm, idx_hbm, xe0_v, xe1_v, xe0h_v, xe1h_v, ct_v,
                 i0_v, i1_v, i2_v, i3_v, i4_v):
        wid = lax.axis_index("s") * 2 + lax.axis_index("c")
        base = wid * rpw
        pltpu.sync_copy(xe_hbm.at[pl.ds(base, rpw)], xe0_v)
        pltpu.sync_copy(xe_hbm.at[pl.ds(nrows + base, rpw)], xe1_v)
        pltpu.sync_copy(xe_hbm.at[pl.ds(2 * nrows + base, rpw)], xe0h_v)
        pltpu.sync_copy(xe_hbm.at[pl.ds(3 * nrows + base, rpw)], xe1h_v)
        pltpu.sync_copy(ct_hbm, ct_v)
        inf = jnp.float32(jnp.inf)

        def insert(c, f2, off):
            m0, m1, m2, m3, m4, i0, i1, i2, i3, i4, sx, xe0h, xe1h = c
            fo = f2 * LQ + off * LQ
            p0f = ct_v[pl.ds(fo, LQ)]
            p1f = ct_v[pl.ds(SPL + fo, LQ)]
            scf = ct_v[pl.ds(2 * SPL + fo, LQ)]
            fs = jnp.zeros((LQ,), jnp.int32) + (f2 + off)
            v = (sx + (xe0h * p0f + xe1h * p1f)) + scf
            b0 = v < m0
            b1 = v < m1
            b2 = v < m2
            b3 = v < m3
            b4 = v < m4
            n0 = jnp.minimum(m0, v)
            n1 = jnp.minimum(m1, jnp.maximum(m0, v))
            n2 = jnp.minimum(m2, jnp.maximum(m1, v))
            n3 = jnp.minimum(m3, jnp.maximum(m2, v))
            n4 = jnp.minimum(m4, jnp.maximum(m3, v))
            j0 = jnp.where(b0, fs, i0)
            j1 = jnp.where(b1, jnp.where(b0, i0, fs), i1)
            j2 = jnp.where(b2, jnp.where(b1, i1, fs), i2)
            j3 = jnp.where(b3, jnp.where(b2, i2, fs), i3)
            j4 = jnp.where(b4, jnp.where(b3, i3, fs), i4)
            return (n0, n1, n2, n3, n4, j0, j1, j2, j3, j4, sx, xe0h, xe1h)

        def group_body(g, carry_out):
            gsl = pl.ds(g * LQ, LQ)
            xe0 = xe0_v[gsl]
            xe1 = xe1_v[gsl]
            xe0h = xe0h_v[gsl]
            xe1h = xe1h_v[gsl]
            sx = xe0 * xe0 + xe1 * xe1

            def fcn_body(fi, c):
                return insert(insert(c, fi * 2, 0), fi * 2, 1)

            init = tuple(jnp.full((LQ,), inf) for _ in range(K)) + \
                tuple(jnp.zeros((LQ,), jnp.int32) for _ in range(K)) + \
                (sx, xe0h, xe1h)
            res = lax.fori_loop(0, N_FCNS // 2, fcn_body, init)
            sl = pl.ds(g * LQ, LQ)
            i0_v[sl] = res[5]
            i1_v[sl] = res[6]
            i2_v[sl] = res[7]
            i3_v[sl] = res[8]
            i4_v[sl] = res[9]
            return carry_out

        lax.fori_loop(0, ng, group_body, 0)
        pltpu.sync_copy(i0_v, idx_hbm.at[pl.ds(base, rpw)])
        pltpu.sync_copy(i1_v, idx_hbm.at[pl.ds(nrows + base, rpw)])
        pltpu.sync_copy(i2_v, idx_hbm.at[pl.ds(2 * nrows + base, rpw)])
        pltpu.sync_copy(i3_v, idx_hbm.at[pl.ds(3 * nrows + base, rpw)])
        pltpu.sync_copy(i4_v, idx_hbm.at[pl.ds(4 * nrows + base, rpw)])

    return _sc_body


def _combine_body(idx_ref, xe8_ref, w4_ref, off_ref, ctrs_ref, out_ref):
    nb = out_ref.shape[0]
    f32 = jnp.float32
    fio = jax.lax.broadcasted_iota(jnp.int32, (NFP, nb), 0)
    eq = (fio == idx_ref[0:1, :])
    for k in range(1, K):
        eq = eq | (fio == idx_ref[k:k + 1, :])
    m = eq.astype(f32)
    w00 = w4_ref[:, 0:1]
    w01 = w4_ref[:, 1:2]
    w10 = w4_ref[:, 2:3]
    w11 = w4_ref[:, 3:4]
    o0 = off_ref[:, 0:1]
    o1 = off_ref[:, 1:2]
    cc0 = ctrs_ref[:, 0:1]
    cc1 = ctrs_ref[:, 1:2]
    zc = jnp.zeros((N_FCNS, 1), f32)
    cmat = jnp.concatenate(
        [o0 - cc0 * w00 - cc1 * w10, o1 - cc0 * w01 - cc1 * w11,
         w00, w01, w10, w11, zc, zc], axis=1)
    cmat = jnp.concatenate([cmat, jnp.zeros((NFP - N_FCNS, 8), f32)], axis=0)
    r = _dot(m, cmat, precision=jax.lax.Precision.HIGHEST,
             dims=(((0,), (0,)), ((), ())))
    xe0 = xe8_ref[:, 0:1]
    xe1 = xe8_ref[:, 1:2]
    out0 = r[:, 0:1] + xe0 * r[:, 2:3] + xe1 * r[:, 4:5]
    out1 = r[:, 1:2] + xe0 * r[:, 3:4] + xe1 * r[:, 5:6]
    out_ref[:] = jnp.concatenate([out0, out1], axis=1)


def _half_pipeline(x, half, ws, out_w, ob2, ctrs, w4, offsets):
    nrows = NHALF
    rpw = nrows // NW
    off = half * (nrows // BLK)
    xe8, xet, ctspl = pl.pallas_call(
        _dense_body,
        grid=(nrows // BLK,),
        in_specs=[
            pl.BlockSpec((BLK, D_IN), lambda i, o=off: (i + o, 0)),
            pl.BlockSpec((D_IN, GROWTH), lambda i: (0, 0)),
            pl.BlockSpec((D_IN + GROWTH, GROWTH), lambda i: (0, 0)),
            pl.BlockSpec((D_IN + 2 * GROWTH, GROWTH), lambda i: (0, 0)),
            pl.BlockSpec((D_IN + 3 * GROWTH, GROWTH), lambda i: (0, 0)),
            pl.BlockSpec((D_IN + 4 * GROWTH, GROWTH), lambda i: (0, 0)),
            pl.BlockSpec((D_IN + 5 * GROWTH, D_EMBED), lambda i: (0, 0)),
            pl.BlockSpec((1, D_EMBED), lambda i: (0, 0)),
            pl.BlockSpec((N_FCNS, D_EMBED), lambda i: (0, 0)),
        ],
        out_specs=[
            pl.BlockSpec((BLK, 8), lambda i: (i, 0)),
            pl.BlockSpec((8, BLK), lambda i: (0, i)),
            pl.BlockSpec((3 * NFP, LQ), lambda i: (0, 0)),
        ],
        out_shape=[
            jax.ShapeDtypeStruct((nrows, 8), jnp.float32),
            jax.ShapeDtypeStruct((8, nrows), jnp.float32),
            jax.ShapeDtypeStruct((3 * NFP, LQ), jnp.float32),
        ],
        scratch_shapes=[pltpu.VMEM((BLK, HW), jnp.float32),
                        pltpu.VMEM((HW, 48), jnp.float32)],
    )(x, *ws, out_w, ob2, ctrs)

    mesh = plsc.VectorSubcoreMesh(core_axis_name="c", subcore_axis_name="s")
    sc_knn = functools.partial(
        pl.kernel,
        mesh=mesh,
        out_type=jax.ShapeDtypeStruct((8 * nrows,), jnp.int32),
        scratch_types=[
            pltpu.VMEM((rpw,), jnp.float32),
            pltpu.VMEM((rpw,), jnp.float32),
            pltpu.VMEM((rpw,), jnp.float32),
            pltpu.VMEM((rpw,), jnp.float32),
            pltpu.VMEM((3 * SPL,), jnp.float32),
            pltpu.VMEM((rpw,), jnp.int32),
            pltpu.VMEM((rpw,), jnp.int32),
            pltpu.VMEM((rpw,), jnp.int32),
            pltpu.VMEM((rpw,), jnp.int32),
            pltpu.VMEM((rpw,), jnp.int32),
        ],
    )(_make_sc_body(nrows))
    idx = sc_knn(xet.reshape(8 * nrows), ctspl.reshape(3 * SPL)).reshape(8, nrows)

    return pl.pallas_call(
        _combine_body,
        grid=(nrows // BLK,),
        in_specs=[
            pl.BlockSpec((8, BLK), lambda i: (0, i)),
            pl.BlockSpec((BLK, 8), lambda i: (i, 0)),
            pl.BlockSpec((N_FCNS, 4), lambda i: (0, 0)),
            pl.BlockSpec((N_FCNS, D_EMBED), lambda i: (0, 0)),
            pl.BlockSpec((N_FCNS, D_EMBED), lambda i: (0, 0)),
        ],
        out_specs=pl.BlockSpec((BLK, D_OUT), lambda i: (i, 0)),
        out_shape=jax.ShapeDtypeStruct((nrows, D_OUT), jnp.float32),
    )(idx, xe8, w4, offsets, ctrs)


@jax.jit
def kernel(x, dense_w0, dense_w1, dense_w2, dense_w3, dense_w4, out_w, out_b,
           ctrs, wts, offsets):
    ws = (dense_w0, dense_w1, dense_w2, dense_w3, dense_w4)
    ob2 = out_b.reshape(1, D_EMBED)
    w4 = wts.reshape(N_FCNS, 4)
    out_a = _half_pipeline(x, 0, ws, out_w, ob2, ctrs, w4, offsets)
    out_b_ = _half_pipeline(x, 1, ws, out_w, ob2, ctrs, w4, offsets)
    return jnp.concatenate([out_a, out_b_], axis=0)


# speedup vs baseline: 1.0097x; 1.0097x over previous
import functools

import jax
import jax.numpy as jnp
from jax import lax
from jax.experimental import pallas as pl
from jax.experimental.pallas import tpu as pltpu
from jax.experimental.pallas import tpu_sc as plsc

N_SMPS = 16384
D_IN = 128
N_LAYERS = 5
GROWTH = 5
D_EMBED = 2
N_FCNS = 250
K = 5
D_OUT = 2

HW = 160
NFP = 256
BLK = 4096

NW = 32
LQ = 16
SPL = NFP * LQ
NHALF = N_SMPS // 2


def _dot(a, b, precision=None, dims=(((1,), (0,)), ((), ()))):
    return jax.lax.dot_general(a, b, dims, precision=precision,
                               preferred_element_type=jnp.float32)


def _dense_body(x_ref, w0_ref, w1_ref, w2_ref, w3_ref, w4_ref, ow_ref, ob_ref,
                ctrs_ref, xe8_ref, xet_ref, ctspl_ref, hbuf, wcat):
    nb = x_ref.shape[0]
    bf = jnp.bfloat16
    f32 = jnp.float32

    @pl.when(pl.program_id(0) == 0)
    def _():
        wcat[:, :] = jnp.zeros((HW, 48), f32)
        wcat[:D_IN, 0:GROWTH] = w0_ref[:]
        wcat[:D_IN + GROWTH, 8:8 + GROWTH] = w1_ref[:]
        wcat[:D_IN + 2 * GROWTH, 16:16 + GROWTH] = w2_ref[:]
        wcat[:D_IN + 3 * GROWTH, 24:24 + GROWTH] = w3_ref[:]
        wcat[:D_IN + 4 * GROWTH, 32:32 + GROWTH] = w4_ref[:]
        wcat[:D_IN + 5 * GROWTH, 40:40 + D_EMBED] = ow_ref[:]

    hbuf[:, :D_IN] = x_ref[:]
    hbuf[:, D_IN:] = jnp.zeros((nb, HW - D_IN), f32)
    for i in range(N_LAYERS):
        w = wcat[:, 8 * i:8 * i + 8]
        z = _dot(hbuf[:].astype(bf), w.astype(bf))
        t = jnp.tanh(z).astype(bf).astype(f32)
        if i == 0:
            hbuf[:, :D_IN] = x_ref[:].astype(bf).astype(f32)
        hbuf[:, D_IN + GROWTH * i:D_IN + GROWTH * i + 8] = t
    xe2 = _dot(hbuf[:].astype(bf), wcat[:, 40:48].astype(bf))
    xe2 = xe2 + jnp.concatenate(
        [ob_ref[:], jnp.zeros((1, 6), f32)], axis=1)
    xe0 = xe2[:, 0:1]
    xe1 = xe2[:, 1:2]
    xe0h = xe0.astype(bf).astype(f32)
    xe1h = xe1.astype(bf).astype(f32)
    zc = jnp.zeros((nb, 1), f32)
    xe8 = jnp.concatenate([xe0, xe1, xe0h, xe1h, zc, zc, zc, zc], axis=1)
    xe8_ref[:] = xe8
    xet_ref[:] = jax.lax.transpose(xe8, (1, 0))

    @pl.when(pl.program_id(0) == 0)
    def _():
        cT = ctrs_ref[:]
        c0 = cT[:, 0:1]
        c1 = cT[:, 1:2]
        sc = c0 * c0 + c1 * c1
        cols = jnp.concatenate(
            [-2.0 * c0.astype(bf).astype(f32),
             -2.0 * c1.astype(bf).astype(f32), sc],
            axis=1)
        pad = jnp.concatenate(
            [jnp.zeros((NFP - N_FCNS, 2), f32),
             jnp.full((NFP - N_FCNS, 1), jnp.inf, f32)], axis=1)
        colsp = jnp.concatenate([cols, pad], axis=0)
        ctspl_ref[:] = jnp.concatenate(
            [jnp.broadcast_to(colsp[:, p:p + 1], (NFP, LQ)) for p in range(3)],
            axis=0)


def _make_sc_body(nrows):
    rpw = nrows // NW
    ng = rpw // LQ

    def _sc_body(xe_hbm, ct_hbm, idx_hbm, xe0_v, xe1_v, xe0h_v, xe1h_v, ct_v,
                 i0_v, i1_v, i2_v, i3_v, i4_v):
        wid = lax.axis_index("s") * 2 + lax.axis_index("c")
        base = wid * rpw
        pltpu.sync_copy(xe_hbm.at[pl.ds(base, rpw)], xe0_v)
        pltpu.sync_copy(xe_hbm.at[pl.ds(nrows + base, rpw)], xe1_v)
        pltpu.sync_copy(xe_hbm.at[pl.ds(2 * nrows + base, rpw)], xe0h_v)
        pltpu.sync_copy(xe_hbm.at[pl.ds(3 * nrows + base, rpw)], xe1h_v)
        pltpu.sync_copy(ct_hbm, ct_v)
        inf = jnp.float32(jnp.inf)

        def insert(c, f2, off):
            m0, m1, m2, m3, m4, i0, i1, i2, i3, i4, sx, xe0h, xe1h = c
            fo = f2 * LQ + off * LQ
            p0f = ct_v[pl.ds(fo, LQ)]
            p1f = ct_v[pl.ds(SPL + fo, LQ)]
            scf = ct_v[pl.ds(2 * SPL + fo, LQ)]
            fs = jnp.zeros((LQ,), jnp.int32) + (f2 + off)
            v = (sx + (xe0h * p0f + xe1h * p1f)) + scf
            b0 = v < m0
            b1 = v < m1
            b2 = v < m2
            b3 = v < m3
            b4 = v < m4
            n0 = jnp.minimum(m0, v)
            n1 = jnp.minimum(m1, jnp.maximum(m0, v))
            n2 = jnp.minimum(m2, jnp.maximum(m1, v))
            n3 = jnp.minimum(m3, jnp.maximum(m2, v))
            n4 = jnp.minimum(m4, jnp.maximum(m3, v))
            j0 = jnp.where(b0, fs, i0)
            j1 = jnp.where(b1, jnp.where(b0, i0, fs), i1)
            j2 = jnp.where(b2, jnp.where(b1, i1, fs), i2)
            j3 = jnp.where(b3, jnp.where(b2, i2, fs), i3)
            j4 = jnp.where(b4, jnp.where(b3, i3, fs), i4)
            return (n0, n1, n2, n3, n4, j0, j1, j2, j3, j4, sx, xe0h, xe1h)

        def group_body(g, carry_out):
            gsl = pl.ds(g * LQ, LQ)
            xe0 = xe0_v[gsl]
            xe1 = xe1_v[gsl]
            xe0h = xe0h_v[gsl]
            xe1h = xe1h_v[gsl]
            sx = xe0 * xe0 + xe1 * xe1

            def fcn_body(fi, c):
                return insert(insert(c, fi * 2, 0), fi * 2, 1)

            init = tuple(jnp.full((LQ,), inf) for _ in range(K)) + \
                tuple(jnp.zeros((LQ,), jnp.int32) for _ in range(K)) + \
                (sx, xe0h, xe1h)
            res = lax.fori_loop(0, N_FCNS // 2, fcn_body, init)
            sl = pl.ds(g * LQ, LQ)
            i0_v[sl] = res[5]
            i1_v[sl] = res[6]
            i2_v[sl] = res[7]
            i3_v[sl] = res[8]
            i4_v[sl] = res[9]
            return carry_out

        lax.fori_loop(0, ng, group_body, 0)
        pltpu.sync_copy(i0_v, idx_hbm.at[pl.ds(base, rpw)])
        pltpu.sync_copy(i1_v, idx_hbm.at[pl.ds(nrows + base, rpw)])
        pltpu.sync_copy(i2_v, idx_hbm.at[pl.ds(2 * nrows + base, rpw)])
        pltpu.sync_copy(i3_v, idx_hbm.at[pl.ds(3 * nrows + base, rpw)])
        pltpu.sync_copy(i4_v, idx_hbm.at[pl.ds(4 * nrows + base, rpw)])

    return _sc_body


def _combine_body(idx_ref, xe8_ref, w4_ref, off_ref, ctrs_ref, out_ref):
    nb = out_ref.shape[0]
    f32 = jnp.float32
    fio = jax.lax.broadcasted_iota(jnp.int32, (NFP, nb), 0)
    eq = (fio == idx_ref[0:1, :])
    for k in range(1, K):
        eq = eq | (fio == idx_ref[k:k + 1, :])
    m = eq.astype(f32)
    w00 = w4_ref[:, 0:1]
    w01 = w4_ref[:, 1:2]
    w10 = w4_ref[:, 2:3]
    w11 = w4_ref[:, 3:4]
    o0 = off_ref[:, 0:1]
    o1 = off_ref[:, 1:2]
    cc0 = ctrs_ref[:, 0:1]
    cc1 = ctrs_ref[:, 1:2]
    zc = jnp.zeros((N_FCNS, 1), f32)
    cmat = jnp.concatenate(
        [o0 - cc0 * w00 - cc1 * w10, o1 - cc0 * w01 - cc1 * w11,
         w00, w01, w10, w11, zc, zc], axis=1)
    cmat = jnp.concatenate([cmat, jnp.zeros((NFP - N_FCNS, 8), f32)], axis=0)
    r = _dot(m, cmat, precision=jax.lax.Precision.HIGHEST,
             dims=(((0,), (0,)), ((), ())))
    xe0 = xe8_ref[:, 0:1]
    xe1 = xe8_ref[:, 1:2]
    out0 = r[:, 0:1] + xe0 * r[:, 2:3] + xe1 * r[:, 4:5]
    out1 = r[:, 1:2] + xe0 * r[:, 3:4] + xe1 * r[:, 5:6]
    out_ref[:] = jnp.concatenate([out0, out1], axis=1)


def _half_pipeline(x, half, ws, out_w, ob2, ctrs, w4, offsets):
    nrows = NHALF
    rpw = nrows // NW
    off = half * (nrows // BLK)
    xe8, xet, ctspl = pl.pallas_call(
        _dense_body,
        grid=(nrows // BLK,),
        in_specs=[
            pl.BlockSpec((BLK, D_IN), lambda i, o=off: (i + o, 0)),
            pl.BlockSpec((D_IN, GROWTH), lambda i: (0, 0)),
            pl.BlockSpec((D_IN + GROWTH, GROWTH), lambda i: (0, 0)),
            pl.BlockSpec((D_IN + 2 * GROWTH, GROWTH), lambda i: (0, 0)),
            pl.BlockSpec((D_IN + 3 * GROWTH, GROWTH), lambda i: (0, 0)),
            pl.BlockSpec((D_IN + 4 * GROWTH, GROWTH), lambda i: (0, 0)),
            pl.BlockSpec((D_IN + 5 * GROWTH, D_EMBED), lambda i: (0, 0)),
            pl.BlockSpec((1, D_EMBED), lambda i: (0, 0)),
            pl.BlockSpec((N_FCNS, D_EMBED), lambda i: (0, 0)),
        ],
        out_specs=[
            pl.BlockSpec((BLK, 8), lambda i: (i, 0)),
            pl.BlockSpec((8, BLK), lambda i: (0, i)),
            pl.BlockSpec((3 * NFP, LQ), lambda i: (0, 0)),
        ],
        out_shape=[
            jax.ShapeDtypeStruct((nrows, 8), jnp.float32),
            jax.ShapeDtypeStruct((8, nrows), jnp.float32),
            jax.ShapeDtypeStruct((3 * NFP, LQ), jnp.float32),
        ],
        scratch_shapes=[pltpu.VMEM((BLK, HW), jnp.float32),
                        pltpu.VMEM((HW, 48), jnp.float32)],
    )(x, *ws, out_w, ob2, ctrs)

    mesh = plsc.VectorSubcoreMesh(core_axis_name="c", subcore_axis_name="s")
    sc_knn = functools.partial(
        pl.kernel,
        mesh=mesh,
        out_type=jax.ShapeDtypeStruct((8 * nrows,), jnp.int32),
        scratch_types=[
            pltpu.VMEM((rpw,), jnp.float32),
            pltpu.VMEM((rpw,), jnp.float32),
            pltpu.VMEM((rpw,), jnp.float32),
            pltpu.VMEM((rpw,), jnp.float32),
            pltpu.VMEM((3 * SPL,), jnp.float32),
            pltpu.VMEM((rpw,), jnp.int32),
            pltpu.VMEM((rpw,), jnp.int32),
            pltpu.VMEM((rpw,), jnp.int32),
            pltpu.VMEM((rpw,), jnp.int32),
            pltpu.VMEM((rpw,), jnp.int32),
        ],
    )(_make_sc_body(nrows))
    idx = sc_knn(xet.reshape(8 * nrows), ctspl.reshape(3 * SPL)).reshape(8, nrows)

    return pl.pallas_call(
        _combine_body,
        grid=(nrows // BLK,),
        in_specs=[
            pl.BlockSpec((8, BLK), lambda i: (0, i)),
            pl.BlockSpec((BLK, 8), lambda i: (i, 0)),
            pl.BlockSpec((N_FCNS, 4), lambda i: (0, 0)),
            pl.BlockSpec((N_FCNS, D_EMBED), lambda i: (0, 0)),
            pl.BlockSpec((N_FCNS, D_EMBED), lambda i: (0, 0)),
        ],
        out_specs=pl.BlockSpec((BLK, D_OUT), lambda i: (i, 0)),
        out_shape=jax.ShapeDtypeStruct((nrows, D_OUT), jnp.float32),
    )(idx, xe8, w4, offsets, ctrs)


@jax.jit
def kernel(x, dense_w0, dense_w1, dense_w2, dense_w3, dense_w4, out_w, out_b,
           ctrs, wts, offsets):
    ws = (dense_w0, dense_w1, dense_w2, dense_w3, dense_w4)
    ob2 = out_b.reshape(1, D_EMBED)
    w4 = wts.reshape(N_FCNS, 4)
    out_a = _half_pipeline(x, 0, ws, out_w, ob2, ctrs, w4, offsets)
    out_b_ = _half_pipeline(x, 1, ws, out_w, ob2, ctrs, w4, offsets)
    return jnp.concatenate([out_a, out_b_], axis=0)
